# Initial kernel scaffold; baseline (speedup 1.0000x reference)
import jax
import jax.numpy as jnp
from jax.experimental import pallas as pl


def kernel(inputs, states, Win, Wres):
    raise NotImplementedError("write your pallas kernel here")


# speedup vs baseline: 1.2095x; 1.2095x over previous
import jax
import jax.numpy as jnp
from jax.experimental import pallas as pl

_B = 1024
_S = 4096
_I = 256
_BJ = 512


def _esn_tile(inputs_ref, states_ref, win_ref, wres_ref, states_j_ref, out_ref):
    sb = states_ref[...].astype(jnp.bfloat16)
    wb = wres_ref[...].astype(jnp.bfloat16)
    ib = inputs_ref[...].astype(jnp.bfloat16)
    winb = win_ref[...].astype(jnp.bfloat16)
    z = jnp.dot(sb, wb, preferred_element_type=jnp.float32)
    z = z + jnp.dot(ib, winb, preferred_element_type=jnp.float32)
    cand = jnp.tanh(z)
    sj = states_j_ref[...]
    out_ref[...] = sj + (cand - sj)


def kernel(inputs, states, Win, Wres):
    grid = (_S // _BJ,)
    return pl.pallas_call(
        _esn_tile,
        grid=grid,
        in_specs=[
            pl.BlockSpec((_B, _I), lambda j: (0, 0)),
            pl.BlockSpec((_B, _S), lambda j: (0, 0)),
            pl.BlockSpec((_I, _BJ), lambda j: (0, j)),
            pl.BlockSpec((_S, _BJ), lambda j: (0, j)),
            pl.BlockSpec((_B, _BJ), lambda j: (0, j)),
        ],
        out_specs=pl.BlockSpec((_B, _BJ), lambda j: (0, j)),
        out_shape=jax.ShapeDtypeStruct((_B, _S), jnp.float32),
    )(inputs, states, Win, Wres, states)


# speedup vs baseline: 1.2773x; 1.0561x over previous
import jax
import jax.numpy as jnp
from jax.experimental import pallas as pl
from jax.experimental.pallas import tpu as pltpu

_B = 1024
_S = 4096
_I = 256
_BJ = 512


def _esn_tile(inputs_ref, states_ref, win_ref, wres_ref, out_ref, sb_ref):
    j = pl.program_id(0)

    @pl.when(j == 0)
    def _cast_states():
        sb_ref[...] = states_ref[...].astype(jnp.bfloat16)

    wb = wres_ref[...].astype(jnp.bfloat16)
    ib = inputs_ref[...].astype(jnp.bfloat16)
    winb = win_ref[...].astype(jnp.bfloat16)
    z = jnp.dot(sb_ref[...], wb, preferred_element_type=jnp.float32)
    z = z + jnp.dot(ib, winb, preferred_element_type=jnp.float32)
    cand = jnp.tanh(z)
    sj = states_ref[:, pl.ds(j * _BJ, _BJ)]
    out_ref[...] = sj + (cand - sj)


def kernel(inputs, states, Win, Wres):
    grid = (_S // _BJ,)
    return pl.pallas_call(
        _esn_tile,
        grid=grid,
        in_specs=[
            pl.BlockSpec((_B, _I), lambda j: (0, 0)),
            pl.BlockSpec((_B, _S), lambda j: (0, 0)),
            pl.BlockSpec((_I, _BJ), lambda j: (0, j)),
            pl.BlockSpec((_S, _BJ), lambda j: (0, j)),
        ],
        out_specs=pl.BlockSpec((_B, _BJ), lambda j: (0, j)),
        out_shape=jax.ShapeDtypeStruct((_B, _S), jnp.float32),
        scratch_shapes=[pltpu.VMEM((_B, _S), jnp.bfloat16)],
    )(inputs, states, Win, Wres)
